# baseline (speedup 1.0000x reference)
import functools

import jax
import jax.numpy as jnp
from jax import lax
from jax.experimental import pallas as pl
from jax.experimental.pallas import tpu as pltpu
from jax.experimental.pallas import tpu_sc as plsc

_NTYPES = ["stock", "financial", "macro", "news", "policy"]
_IN_DIM = {"stock": 5, "financial": 16, "macro": 21, "news": 768, "policy": 768}
_REL = [("stock", "financial"), ("stock", "macro"), ("stock", "news"), ("stock", "policy"),
        ("financial", "macro"), ("financial", "stock"), ("financial", "news"), ("financial", "policy"),
        ("macro", "stock"), ("macro", "financial"), ("macro", "news"), ("macro", "policy"),
        ("news", "stock"), ("news", "financial"), ("news", "macro"), ("news", "policy"),
        ("policy", "stock"), ("policy", "macro"), ("policy", "news"), ("policy", "financial")]
_TI = {nt: i for i, nt in enumerate(_NTYPES)}
_N = 10000
_E = 60000
_HID = 64
_NC = 10
_NREL = 20
_L = 16
_CHA = 12000
_NCHA = _E // _CHA
_CHB = 6000
_NCHB = _E // _CHB
_NV = _N // _L

_DST_TI = [_TI[d] for (_, d) in _REL]

def _rsqrt16(x):
    xi = plsc.bitcast(x, jnp.int32)
    yi = jnp.int32(0x5F3759DF) - lax.shift_right_arithmetic(xi, 1)
    y = plsc.bitcast(yi, jnp.float32)
    for _ in range(3):
        y = y * (1.5 - 0.5 * x * y * y)
    return y


def _sc_kernel_a_body(ei_hbm, a_out, c_out, w_out, sw_out,
                      src0, dst0, src1, dst1, cnt_o, cnt_i, t_v, sw_v,
                      sem0, sem1):
    wid = lax.axis_index("s") * 2 + lax.axis_index("c")

    @pl.when(wid < _NREL)
    def _():
        r = wid
        zeros16 = jnp.zeros((_L,), jnp.float32)
        ones16 = jnp.ones((_L,), jnp.float32)
        bufs = [(src0, dst0), (src1, dst1)]
        sems = [sem0, sem1]
        descs = [None, None]

        def start(k):
            b = k % 2
            descs[b] = (
                pltpu.async_copy(ei_hbm.at[r, 0, pl.ds(k * _CHA, _CHA)],
                                 bufs[b][0], sems[b]),
                pltpu.async_copy(ei_hbm.at[r, 1, pl.ds(k * _CHA, _CHA)],
                                 bufs[b][1], sems[b]),
            )

        def edge_pass(body_fn):
            for k in range(_NCHA):
                b = k % 2
                d1, d2 = descs[b]
                d1.wait()
                d2.wait()
                if k + 1 < _NCHA:
                    start(k + 1)
                body_fn(bufs[b][0], bufs[b][1])

        start(0)

        @plsc.parallel_loop(0, _N, step=_L, unroll=8)
        def _(i):
            sl = pl.ds(i, _L)
            cnt_o[sl] = zeros16
            cnt_i[sl] = zeros16
            t_v[sl] = zeros16

        def deg_body(sv, dv):
            @plsc.parallel_loop(0, _CHA, step=_L, unroll=8)
            def _(i):
                sl = pl.ds(i, _L)
                plsc.addupdate_scatter(cnt_o, [sv[sl]], ones16)
                plsc.addupdate_scatter(cnt_i, [dv[sl]], ones16)
        edge_pass(deg_body)

        start(0)

        @plsc.parallel_loop(0, _N, step=_L, unroll=4)
        def _(i):
            sl = pl.ds(i, _L)
            cnt_o[sl] = _rsqrt16(jnp.maximum(cnt_o[sl], 1.0))
            cnt_i[sl] = _rsqrt16(jnp.maximum(cnt_i[sl], 1.0))

        def t_body(sv, dv):
            @plsc.parallel_loop(0, _CHA, step=_L, unroll=8)
            def _(i):
                sl = pl.ds(i, _L)
                cv = plsc.load_gather(cnt_i, [dv[sl]])
                plsc.addupdate_scatter(t_v, [sv[sl]], cv)
        edge_pass(t_body)

        @plsc.parallel_loop(0, _N, step=_L, unroll=4, carry=zeros16)
        def sw_final(i, acc):
            sl = pl.ds(i, _L)
            wv = t_v[sl] * cnt_o[sl]
            t_v[sl] = wv
            return acc + wv
        sw_v[...] = sw_final

        pltpu.sync_copy(cnt_o, a_out.at[r])
        pltpu.sync_copy(cnt_i, c_out.at[r])
        pltpu.sync_copy(t_v, w_out.at[r])
        pltpu.sync_copy(sw_v, sw_out.at[r])


def _sc_kernel_b_body(ei_hbm, a_all, c_all, w_all, v_out,
                      src0, dst0, src1, dst1, c_v, a_v,
                      w0, w1, w2, w3, u0, u1, u2, u3, sem0, sem1):
    wid = lax.axis_index("s") * 2 + lax.axis_index("c")

    @pl.when(wid < _NREL)
    def _():
        r1 = wid
        ws = [w0, w1, w2, w3]
        us = [u0, u1, u2, u3]
        zeros16 = jnp.zeros((_L,), jnp.float32)
        bufs = [(src0, dst0), (src1, dst1)]
        sems = [sem0, sem1]
        descs = [None, None]

        def start(k):
            b = k % 2
            descs[b] = (
                pltpu.async_copy(ei_hbm.at[r1, 0, pl.ds(k * _CHB, _CHB)],
                                 bufs[b][0], sems[b]),
                pltpu.async_copy(ei_hbm.at[r1, 1, pl.ds(k * _CHB, _CHB)],
                                 bufs[b][1], sems[b]),
            )
        start(0)

        r2b = jnp.int32(0)
        for rr in range(_NREL):
            r2b = jnp.where(wid == rr, jnp.int32(4 * _DST_TI[rr]), r2b)

        pltpu.sync_copy(c_all.at[r1], c_v)
        pltpu.sync_copy(a_all.at[r1], a_v)
        for j in range(4):
            pltpu.sync_copy(w_all.at[r2b + j], ws[j])

        @plsc.parallel_loop(0, _N, step=_L, unroll=4)
        def _(i):
            sl = pl.ds(i, _L)
            cv = c_v[sl]
            for j in range(4):
                ws[j][sl] = ws[j][sl] * cv
                us[j][sl] = zeros16

        for k in range(_NCHB):
            b = k % 2
            d1, d2 = descs[b]
            d1.wait()
            d2.wait()
            if k + 1 < _NCHB:
                start(k + 1)
            sv, dv = bufs[b]

            @plsc.parallel_loop(0, _CHB, step=_L, unroll=8)
            def _(i):
                sl = pl.ds(i, _L)
                s16 = sv[sl]
                d16 = dv[sl]
                for j in range(4):
                    zv = plsc.load_gather(ws[j], [d16])
                    plsc.addupdate_scatter(us[j], [s16], zv)

        @plsc.parallel_loop(0, _N, step=_L, unroll=4)
        def _(i):
            sl = pl.ds(i, _L)
            av = a_v[sl]
            for j in range(4):
                us[j][sl] = us[j][sl] * av

        for j in range(4):
            for k in range(_NSTEP):
                pltpu.sync_copy(us[j].at[pl.ds(k * _CN, _CN)],
                                v_out.at[k, 4 * r1 + j])


@functools.cache
def _sc_kernels():
    mesh = plsc.VectorSubcoreMesh(core_axis_name="c", subcore_axis_name="s",
                                  num_cores=2, num_subcores=16)
    nvec = pltpu.VMEM((_N,), jnp.float32)
    eva = pltpu.VMEM((_CHA,), jnp.int32)
    evb = pltpu.VMEM((_CHB,), jnp.int32)
    dma = pltpu.SemaphoreType.DMA
    cparams = pltpu.CompilerParams(use_tc_tiling_on_sc=False,
                                   needs_layout_passes=False)
    kernel_a = pl.kernel(
        _sc_kernel_a_body,
        out_type=(jax.ShapeDtypeStruct((_NREL, _N), jnp.float32),
                  jax.ShapeDtypeStruct((_NREL, _N), jnp.float32),
                  jax.ShapeDtypeStruct((_NREL, _N), jnp.float32),
                  jax.ShapeDtypeStruct((_NREL, _L), jnp.float32)),
        mesh=mesh,
        scratch_types=(eva, eva, eva, eva, nvec, nvec, nvec,
                       pltpu.VMEM((_L,), jnp.float32), dma, dma),
        compiler_params=cparams,
    )
    kernel_b = pl.kernel(
        _sc_kernel_b_body,
        out_type=jax.ShapeDtypeStruct((_NSTEP, 4 * _NREL, _CN), jnp.float32),
        mesh=mesh,
        scratch_types=(evb, evb, evb, evb) + (nvec,) * 10 + (dma, dma),
        compiler_params=cparams,
    )
    return kernel_a, kernel_b


_CN = 1000
_NSTEP = _N // _CN
_DIMS = [_IN_DIM[nt] for nt in _NTYPES]


def _tc_body(v_ref, f0, f1, f2, f3, f4, w1c0, w1c1, w1c2, w1c3, w1c4,
             b1s, w2f, sws, b2s, wfc, bfc, dgl_ref, out_ref,
             x0, x1, x2, x3, x4):
    fs = [f0, f1, f2, f3, f4]
    w1cs = [w1c0, w1c1, w1c2, w1c3, w1c4]
    xs = [x0, x1, x2, x3, x4]
    step = pl.program_id(0)

    @pl.when(step == 0)
    def _():
        for t in range(5):
            xs[t][...] = jnp.zeros_like(xs[t])

    vblk_all = v_ref[0]
    for t in range(5):
        vblk = vblk_all[16 * t:16 * (t + 1), :]
        xs[t][...] += jnp.dot(vblk, fs[t][...],
                              preferred_element_type=jnp.float32,
                              precision=jax.lax.Precision.HIGHEST)

    @pl.when(step == _NSTEP - 1)
    def _():
        q = [jnp.zeros((1, _HID), jnp.float32) for _ in range(_NREL)]
        for t in range(5):
            dim = _DIMS[t]
            xt = xs[t][...]
            for rl in range(4):
                r1 = 4 * t + rl
                w1 = w1cs[t][rl * dim:(rl + 1) * dim, :]
                y4 = jnp.dot(xt[4 * rl:4 * rl + 4, :], w1,
                             preferred_element_type=jnp.float32, precision=jax.lax.Precision.HIGHEST)
                for j in range(4):
                    r2 = 4 * _DST_TI[r1] + j
                    q[r2] = q[r2] + y4[j:j + 1, :]
        b1sum = []
        for st in range(5):
            acc = jnp.zeros((1, _HID), jnp.float32)
            for r1 in range(_NREL):
                if _DST_TI[r1] == st:
                    acc = acc + b1s[r1:r1 + 1, :]
            b1sum.append(acc)
        total = jnp.zeros((1, _HID), jnp.float32)
        for r2 in range(_NREL):
            sw_r2 = jnp.sum(sws[r2:r2 + 1, :])
            qf = q[r2] + sw_r2 * b1sum[r2 // 4]
            total = total + jnp.dot(qf, w2f[_HID * r2:_HID * (r2 + 1), :],
                                    preferred_element_type=jnp.float32, precision=jax.lax.Precision.HIGHEST)
        total = total + _N * jnp.sum(b2s[...], axis=0, keepdims=True)
        dgl = total * (1.0 / (5.0 * _N))
        dgl_ref[...] = dgl
        out_ref[...] = jnp.dot(dgl, wfc[...],
                               preferred_element_type=jnp.float32, precision=jax.lax.Precision.HIGHEST) + bfc[...]


def _tc_call(v_all, feats_l, w1cs, b1s, w2f, sws, b2s, wfc, bfc):
    whole = pl.BlockSpec(index_map=lambda k: (0, 0))
    in_specs = ([pl.BlockSpec((1, 4 * _NREL, _CN), lambda k: (k, 0, 0))]
                + [pl.BlockSpec((_CN, d), lambda k: (k, 0)) for d in _DIMS]
                + [whole] * 5
                + [whole] * 5)
    in_specs.append(whole)
    return pl.pallas_call(
        _tc_body,
        grid=(_NSTEP,),
        in_specs=in_specs,
        out_specs=[pl.BlockSpec((1, _HID), lambda k: (0, 0)),
                   pl.BlockSpec((1, _NC), lambda k: (0, 0))],
        out_shape=[jax.ShapeDtypeStruct((1, _HID), jnp.float32),
                   jax.ShapeDtypeStruct((1, _NC), jnp.float32)],
        scratch_shapes=[pltpu.VMEM((16, d), jnp.float32) for d in _DIMS],
    )(v_all, *feats_l, *w1cs, b1s, w2f, sws, b2s, wfc, bfc)


def kernel(feats, edges, params):
    keys = [s + "_" + d for s, d in _REL]
    ei_all = jnp.stack([edges[k] for k in keys])

    kernel_a, kernel_b = _sc_kernels()
    a_all, c_all, w_all, sws = kernel_a(ei_all)
    v_all = kernel_b(ei_all, a_all, c_all, w_all)

    feats_l = [feats[nt] for nt in _NTYPES]
    w1cs = [jnp.concatenate([params["W1"][keys[4 * t + rl]] for rl in range(4)],
                            axis=0) for t in range(5)]
    b1s = jnp.stack([params["b1"][k] for k in keys])
    w2f = jnp.concatenate([params["W2"][k] for k in keys], axis=0)
    b2s = jnp.stack([params["b2"][k] for k in keys])
    bfc = params["bfc"].reshape(1, _NC)

    dgl_m, output = _tc_call(v_all, feats_l, w1cs, b1s, w2f, sws, b2s,
                             params["Wfc"], bfc)
    return (dgl_m, output)


# speedup vs baseline: 1.0814x; 1.0814x over previous
import functools

import jax
import jax.numpy as jnp
from jax import lax
from jax.experimental import pallas as pl
from jax.experimental.pallas import tpu as pltpu
from jax.experimental.pallas import tpu_sc as plsc

_NTYPES = ["stock", "financial", "macro", "news", "policy"]
_IN_DIM = {"stock": 5, "financial": 16, "macro": 21, "news": 768, "policy": 768}
_REL = [("stock", "financial"), ("stock", "macro"), ("stock", "news"), ("stock", "policy"),
        ("financial", "macro"), ("financial", "stock"), ("financial", "news"), ("financial", "policy"),
        ("macro", "stock"), ("macro", "financial"), ("macro", "news"), ("macro", "policy"),
        ("news", "stock"), ("news", "financial"), ("news", "macro"), ("news", "policy"),
        ("policy", "stock"), ("policy", "macro"), ("policy", "news"), ("policy", "financial")]
_TI = {nt: i for i, nt in enumerate(_NTYPES)}
_N = 10000
_E = 60000
_HID = 64
_NC = 10
_NREL = 20
_L = 16
_CHA = 20000
_NCHA = _E // _CHA
_CHB = 6000
_NCHB = _E // _CHB
_NV = _N // _L

_DST_TI = [_TI[d] for (_, d) in _REL]

def _rsqrt16(x):
    xi = plsc.bitcast(x, jnp.int32)
    yi = jnp.int32(0x5F3759DF) - lax.shift_right_arithmetic(xi, 1)
    y = plsc.bitcast(yi, jnp.float32)
    for _ in range(3):
        y = y * (1.5 - 0.5 * x * y * y)
    return y


def _sc_kernel_a_body(*refs):
    eis = refs[:_NREL]
    a_out, c_out, w_out, sw_out = refs[_NREL:_NREL + 4]
    eb0, eb1, cnt_o, cnt_i, t_v, sw_v, sem0, sem1 = refs[_NREL + 4:]
    wid = lax.axis_index("s") * 2 + lax.axis_index("c")

    @pl.when(wid < _NREL)
    def _():
        r = wid
        zeros16 = jnp.zeros((_L,), jnp.float32)
        ones16 = jnp.ones((_L,), jnp.float32)
        ebs = [eb0, eb1]
        sems = [sem0, sem1]

        def start(k):
            b = k % 2
            for rr in range(_NREL):
                @pl.when(wid == rr)
                def _(rr=rr):
                    pltpu.async_copy(eis[rr].at[:, pl.ds(k * _CHA, _CHA)],
                                     ebs[b], sems[b])

        def edge_pass(body_fn):
            for k in range(_NCHA):
                b = k % 2
                pltpu.make_async_copy(eis[0].at[:, pl.ds(0, _CHA)],
                                      ebs[b], sems[b]).wait()
                if k + 1 < _NCHA:
                    start(k + 1)
                body_fn(ebs[b])

        start(0)

        @plsc.parallel_loop(0, _N, step=_L, unroll=8)
        def _(i):
            sl = pl.ds(i, _L)
            cnt_o[sl] = zeros16
            cnt_i[sl] = zeros16
            t_v[sl] = zeros16

        def deg_body(eb):
            @plsc.parallel_loop(0, _CHA, step=_L, unroll=8)
            def _(i):
                sl = pl.ds(i, _L)
                plsc.addupdate_scatter(cnt_o, [eb[0, sl]], ones16)
                plsc.addupdate_scatter(cnt_i, [eb[1, sl]], ones16)
        edge_pass(deg_body)

        start(0)

        @plsc.parallel_loop(0, _N, step=_L, unroll=4)
        def _(i):
            sl = pl.ds(i, _L)
            cnt_o[sl] = _rsqrt16(jnp.maximum(cnt_o[sl], 1.0))
            cnt_i[sl] = _rsqrt16(jnp.maximum(cnt_i[sl], 1.0))

        def t_body(eb):
            @plsc.parallel_loop(0, _CHA, step=_L, unroll=8)
            def _(i):
                sl = pl.ds(i, _L)
                cv = plsc.load_gather(cnt_i, [eb[1, sl]])
                plsc.addupdate_scatter(t_v, [eb[0, sl]], cv)
        edge_pass(t_body)

        @plsc.parallel_loop(0, _N, step=_L, unroll=4, carry=zeros16)
        def sw_final(i, acc):
            sl = pl.ds(i, _L)
            wv = t_v[sl] * cnt_o[sl]
            t_v[sl] = wv
            return acc + wv
        sw_v[...] = sw_final

        pltpu.sync_copy(cnt_o, a_out.at[r])
        pltpu.sync_copy(cnt_i, c_out.at[r])
        pltpu.sync_copy(t_v, w_out.at[r])
        pltpu.sync_copy(sw_v, sw_out.at[r])


def _sc_kernel_b_body(*refs):
    eis = refs[:_NREL]
    a_all, c_all, w_all, v_out = refs[_NREL:_NREL + 4]
    (eb0, eb1, c_v, a_v, w0, w1, w2, w3,
     u0, u1, u2, u3, sem0, sem1) = refs[_NREL + 4:]
    wid = lax.axis_index("s") * 2 + lax.axis_index("c")

    @pl.when(wid < _NREL)
    def _():
        r1 = wid
        ws = [w0, w1, w2, w3]
        us = [u0, u1, u2, u3]
        zeros16 = jnp.zeros((_L,), jnp.float32)
        ebs = [eb0, eb1]
        sems = [sem0, sem1]

        def start(k):
            b = k % 2
            for rr in range(_NREL):
                @pl.when(wid == rr)
                def _(rr=rr):
                    pltpu.async_copy(eis[rr].at[:, pl.ds(k * _CHB, _CHB)],
                                     ebs[b], sems[b])
        start(0)

        r2b = jnp.int32(0)
        for rr in range(_NREL):
            r2b = jnp.where(wid == rr, jnp.int32(4 * _DST_TI[rr]), r2b)

        pltpu.sync_copy(c_all.at[r1], c_v)
        pltpu.sync_copy(a_all.at[r1], a_v)
        for j in range(4):
            pltpu.sync_copy(w_all.at[r2b + j], ws[j])

        @plsc.parallel_loop(0, _N, step=_L, unroll=4)
        def _(i):
            sl = pl.ds(i, _L)
            cv = c_v[sl]
            for j in range(4):
                ws[j][sl] = ws[j][sl] * cv
                us[j][sl] = zeros16

        for k in range(_NCHB):
            b = k % 2
            pltpu.make_async_copy(eis[0].at[:, pl.ds(0, _CHB)],
                                  ebs[b], sems[b]).wait()
            if k + 1 < _NCHB:
                start(k + 1)
            eb = ebs[b]

            @plsc.parallel_loop(0, _CHB, step=_L, unroll=4)
            def _(i):
                sl = pl.ds(i, _L)
                s16 = eb[0, sl]
                d16 = eb[1, sl]
                for j in range(4):
                    zv = plsc.load_gather(ws[j], [d16])
                    plsc.addupdate_scatter(us[j], [s16], zv)

        @plsc.parallel_loop(0, _N, step=_L, unroll=4)
        def _(i):
            sl = pl.ds(i, _L)
            av = a_v[sl]
            for j in range(4):
                us[j][sl] = us[j][sl] * av

        for j in range(4):
            for k in range(_NSTEP):
                pltpu.sync_copy(us[j].at[pl.ds(k * _CN, _CN)],
                                v_out.at[k, 4 * r1 + j])


@functools.cache
def _sc_kernels():
    mesh = plsc.VectorSubcoreMesh(core_axis_name="c", subcore_axis_name="s",
                                  num_cores=2, num_subcores=16)
    nvec = pltpu.VMEM((_N,), jnp.float32)
    eva = pltpu.VMEM((2, _CHA), jnp.int32)
    evb = pltpu.VMEM((2, _CHB), jnp.int32)
    dma = pltpu.SemaphoreType.DMA
    cparams = pltpu.CompilerParams(use_tc_tiling_on_sc=False,
                                   needs_layout_passes=False)
    kernel_a = pl.kernel(
        _sc_kernel_a_body,
        out_type=(jax.ShapeDtypeStruct((_NREL, _N), jnp.float32),
                  jax.ShapeDtypeStruct((_NREL, _N), jnp.float32),
                  jax.ShapeDtypeStruct((_NREL, _N), jnp.float32),
                  jax.ShapeDtypeStruct((_NREL, _L), jnp.float32)),
        mesh=mesh,
        scratch_types=(eva, eva, nvec, nvec, nvec,
                       pltpu.VMEM((_L,), jnp.float32), dma, dma),
        compiler_params=cparams,
    )
    kernel_b = pl.kernel(
        _sc_kernel_b_body,
        out_type=jax.ShapeDtypeStruct((_NSTEP, 4 * _NREL, _CN), jnp.float32),
        mesh=mesh,
        scratch_types=(evb, evb) + (nvec,) * 10 + (dma, dma),
        compiler_params=cparams,
    )
    return kernel_a, kernel_b


_CN = 1000
_NSTEP = _N // _CN
_DIMS = [_IN_DIM[nt] for nt in _NTYPES]


def _tc_body(v_ref, f0, f1, f2, f3, f4, w1c0, w1c1, w1c2, w1c3, w1c4,
             b1s, w2f, sws, b2s, wfc, bfc, dgl_ref, out_ref,
             x0, x1, x2, x3, x4):
    fs = [f0, f1, f2, f3, f4]
    w1cs = [w1c0, w1c1, w1c2, w1c3, w1c4]
    xs = [x0, x1, x2, x3, x4]
    step = pl.program_id(0)

    @pl.when(step == 0)
    def _():
        for t in range(5):
            xs[t][...] = jnp.zeros_like(xs[t])

    vblk_all = v_ref[0]
    for t in range(5):
        vblk = vblk_all[16 * t:16 * (t + 1), :]
        xs[t][...] += jnp.dot(vblk, fs[t][...],
                              preferred_element_type=jnp.float32,
                              precision=jax.lax.Precision.HIGHEST)

    @pl.when(step == _NSTEP - 1)
    def _():
        q = [jnp.zeros((1, _HID), jnp.float32) for _ in range(_NREL)]
        for t in range(5):
            dim = _DIMS[t]
            xt = xs[t][...]
            for rl in range(4):
                r1 = 4 * t + rl
                w1 = w1cs[t][rl * dim:(rl + 1) * dim, :]
                y4 = jnp.dot(xt[4 * rl:4 * rl + 4, :], w1,
                             preferred_element_type=jnp.float32, precision=jax.lax.Precision.HIGHEST)
                for j in range(4):
                    r2 = 4 * _DST_TI[r1] + j
                    q[r2] = q[r2] + y4[j:j + 1, :]
        b1sum = []
        for st in range(5):
            acc = jnp.zeros((1, _HID), jnp.float32)
            for r1 in range(_NREL):
                if _DST_TI[r1] == st:
                    acc = acc + b1s[r1:r1 + 1, :]
            b1sum.append(acc)
        total = jnp.zeros((1, _HID), jnp.float32)
        for r2 in range(_NREL):
            sw_r2 = jnp.sum(sws[r2:r2 + 1, :])
            qf = q[r2] + sw_r2 * b1sum[r2 // 4]
            total = total + jnp.dot(qf, w2f[_HID * r2:_HID * (r2 + 1), :],
                                    preferred_element_type=jnp.float32, precision=jax.lax.Precision.HIGHEST)
        total = total + _N * jnp.sum(b2s[...], axis=0, keepdims=True)
        dgl = total * (1.0 / (5.0 * _N))
        dgl_ref[...] = dgl
        out_ref[...] = jnp.dot(dgl, wfc[...],
                               preferred_element_type=jnp.float32, precision=jax.lax.Precision.HIGHEST) + bfc[...]


def _tc_call(v_all, feats_l, w1cs, b1s, w2f, sws, b2s, wfc, bfc):
    whole = pl.BlockSpec(index_map=lambda k: (0, 0))
    in_specs = ([pl.BlockSpec((1, 4 * _NREL, _CN), lambda k: (k, 0, 0))]
                + [pl.BlockSpec((_CN, d), lambda k: (k, 0)) for d in _DIMS]
                + [whole] * 5
                + [whole] * 5)
    in_specs.append(whole)
    return pl.pallas_call(
        _tc_body,
        grid=(_NSTEP,),
        in_specs=in_specs,
        out_specs=[pl.BlockSpec((1, _HID), lambda k: (0, 0)),
                   pl.BlockSpec((1, _NC), lambda k: (0, 0))],
        out_shape=[jax.ShapeDtypeStruct((1, _HID), jnp.float32),
                   jax.ShapeDtypeStruct((1, _NC), jnp.float32)],
        scratch_shapes=[pltpu.VMEM((16, d), jnp.float32) for d in _DIMS],
    )(v_all, *feats_l, *w1cs, b1s, w2f, sws, b2s, wfc, bfc)


def kernel(feats, edges, params):
    keys = [s + "_" + d for s, d in _REL]
    edges_l = [edges[k] for k in keys]

    kernel_a, kernel_b = _sc_kernels()
    a_all, c_all, w_all, sws = kernel_a(*edges_l)
    v_all = kernel_b(*edges_l, a_all, c_all, w_all)

    feats_l = [feats[nt] for nt in _NTYPES]
    w1cs = [jnp.concatenate([params["W1"][keys[4 * t + rl]] for rl in range(4)],
                            axis=0) for t in range(5)]
    b1s = jnp.stack([params["b1"][k] for k in keys])
    w2f = jnp.concatenate([params["W2"][k] for k in keys], axis=0)
    b2s = jnp.stack([params["b2"][k] for k in keys])
    bfc = params["bfc"].reshape(1, _NC)

    dgl_m, output = _tc_call(v_all, feats_l, w1cs, b1s, w2f, sws, b2s,
                             params["Wfc"], bfc)
    return (dgl_m, output)
